# baseline (speedup 1.0000x reference)
import functools

import jax
import jax.numpy as jnp
from jax import lax
from jax.experimental import pallas as pl
from jax.experimental.pallas import tpu as pltpu
from jax.experimental.pallas import tpu_sc as plsc

N = 10000
E = 640000
D = 128
DE = 16

NC = 2
NS = 16
NW = NC * NS
EPT = E // NW
KB = 96
NFULL = EPT // KB
KT = EPT - NFULL * KB
NG = NFULL // 2
NG3 = (NFULL - 4) // 3
NP = 10112
NPT = NP // NS
PKT = 2 * KB
TILE_PK = NFULL * PKT + 2 * KT

_mesh = plsc.VectorSubcoreMesh(
    core_axis_name="c", subcore_axis_name="s", num_cores=NC, num_subcores=NS
)


@functools.partial(
    pl.kernel,
    out_type=jax.ShapeDtypeStruct((NC, NP, D), jnp.float32),
    mesh=_mesh,
    scratch_types=[
        pltpu.VMEM((KB,), jnp.int32),
        pltpu.VMEM((KB,), jnp.int32),
        pltpu.VMEM((KT,), jnp.int32),
        pltpu.VMEM((KB, D), jnp.float32),
        pltpu.VMEM((KB, D), jnp.float32),
        pltpu.VMEM((KT, D), jnp.float32),
        pltpu.VMEM_SHARED((NP, D), jnp.float32),
        pltpu.SemaphoreType.DMA,
        pltpu.SemaphoreType.DMA,
        pltpu.SemaphoreType.DMA,
        pltpu.SemaphoreType.DMA,
    ],
)
def _sc_edge_scatter(ea_hbm, pk_hbm, zeros_hbm, out_a, i0, i1, it_,
                     v0, v1, vt, acc, s0, s1, q0, q1):
    c = lax.axis_index("c")
    s = lax.axis_index("s")
    wid = c * NS + s
    pltpu.sync_copy(zeros_hbm.at[pl.ds(s * NPT, NPT)], acc.at[pl.ds(s * NPT, NPT)])
    plsc.subcore_barrier()
    base = wid * EPT
    pkbase = wid * TILE_PK
    ibufs = (i0, i1)
    isems = (q0, q1)
    vbufs = (v0, v1)
    vsems = (s0, s1)

    def fire(ch, b):
        pltpu.async_copy(pk_hbm.at[pl.ds(pkbase + ch * PKT + KB, KB)],
                         ibufs[b], isems[b])
        pltpu.async_copy(ea_hbm.at[pl.ds(base + ch * KB, KB)],
                         vbufs[b], vsems[b])

    def drain(ch, b):
        pltpu.make_async_copy(pk_hbm.at[pl.ds(pkbase + ch * PKT + KB, KB)],
                              ibufs[b], isems[b]).wait()
        pltpu.make_async_copy(ea_hbm.at[pl.ds(base + ch * KB, KB)],
                              vbufs[b], vsems[b]).wait()

    def scatter(ch, b):
        pltpu.sync_copy(vbufs[b], acc.at[ibufs[b]], add=True)

    fire(0, 0)

    def group(gi, carry):
        for b in range(2):
            ch = gi * 2 + b
            fire(ch + 1, 1 - b)
            drain(ch, b)
            scatter(ch, b)
        return carry

    lax.fori_loop(0, NG - 1, group, 0)
    ch0, ch1 = NFULL - 2, NFULL - 1
    fire(ch1, 1)
    drain(ch0, 0)
    scatter(ch0, 0)
    pltpu.async_copy(pk_hbm.at[pl.ds(pkbase + NFULL * PKT + KT, KT)], it_, q0)
    pltpu.async_copy(ea_hbm.at[pl.ds(base + NFULL * KB, KT)], vt, s0)
    drain(ch1, 1)
    scatter(ch1, 1)
    pltpu.make_async_copy(pk_hbm.at[pl.ds(pkbase + NFULL * PKT + KT, KT)],
                          it_, q0).wait()
    pltpu.make_async_copy(ea_hbm.at[pl.ds(base + NFULL * KB, KT)], vt, s0).wait()
    pltpu.sync_copy(vt, acc.at[it_], add=True)

    plsc.subcore_barrier()
    pltpu.sync_copy(acc.at[pl.ds(s * NPT, NPT)], out_a.at[c, pl.ds(s * NPT, NPT)])


@functools.partial(
    pl.kernel,
    out_type=jax.ShapeDtypeStruct((NC, NP, D), jnp.float32),
    mesh=_mesh,
    scratch_types=[
        pltpu.VMEM((PKT,), jnp.int32),
        pltpu.VMEM((PKT,), jnp.int32),
        pltpu.VMEM((PKT,), jnp.int32),
        pltpu.VMEM((KB, D), jnp.float32),
        pltpu.VMEM((KB, D), jnp.float32),
        pltpu.VMEM((KB, D), jnp.float32),
        pltpu.VMEM_SHARED((NP, D), jnp.float32),
        pltpu.SemaphoreType.DMA,
        pltpu.SemaphoreType.DMA,
        pltpu.SemaphoreType.DMA,
        pltpu.SemaphoreType.DMA,
        pltpu.SemaphoreType.DMA,
        pltpu.SemaphoreType.DMA,
    ],
)
def _sc_gather_scatter(feats_hbm, pk_hbm, zeros_hbm, out_hbm,
                       i0, i1, i2, r0, r1, r2, acc,
                       s0, s1, s2, q0, q1, q2):
    c = lax.axis_index("c")
    s = lax.axis_index("s")
    wid = c * NS + s
    pltpu.sync_copy(zeros_hbm.at[pl.ds(s * NPT, NPT)], acc.at[pl.ds(s * NPT, NPT)])
    plsc.subcore_barrier()
    pkbase = wid * TILE_PK
    ibufs = (i0, i1, i2)
    isems = (q0, q1, q2)
    rbufs = (r0, r1, r2)
    rsems = (s0, s1, s2)
    it_ = i1.at[pl.ds(0, 2 * KT)]
    rt = r1.at[pl.ds(0, KT)]

    def fire_idx(ch, b):
        pltpu.async_copy(pk_hbm.at[pl.ds(pkbase + ch * PKT, PKT)],
                         ibufs[b], isems[b])

    def drain_idx(ch, b):
        pltpu.make_async_copy(pk_hbm.at[pl.ds(pkbase + ch * PKT, PKT)],
                              ibufs[b], isems[b]).wait()

    def gather(ch, b):
        pltpu.async_copy(feats_hbm.at[ibufs[b].at[pl.ds(0, KB)]],
                         rbufs[b], rsems[b])

    def drain_rows(ch, b):
        pltpu.make_async_copy(feats_hbm.at[ibufs[b].at[pl.ds(0, KB)]],
                              rbufs[b], rsems[b]).wait()

    def scatter(ch, b):
        pltpu.sync_copy(rbufs[b], acc.at[ibufs[b].at[pl.ds(KB, KB)]], add=True)

    pltpu.async_copy(pk_hbm.at[pl.ds(pkbase, PKT)], i0, q0).wait()
    gather(0, 0)
    fire_idx(1, 1)
    drain_idx(1, 1)
    gather(1, 1)
    fire_idx(2, 2)

    def group(gi, carry):
        for b in range(3):
            t = gi * 3 + b
            drain_idx(t + 2, (b + 2) % 3)
            gather(t + 2, (b + 2) % 3)
            drain_rows(t, b)
            scatter(t, b)
            fire_idx(t + 3, b)
        return carry

    lax.fori_loop(0, NG3, group, 0)
    cA, cB, cC, cD = NFULL - 4, NFULL - 3, NFULL - 2, NFULL - 1
    drain_idx(cC, 2)
    gather(cC, 2)
    drain_rows(cA, 0)
    scatter(cA, 0)
    fire_idx(cD, 0)
    drain_idx(cD, 0)
    gather(cD, 0)
    drain_rows(cB, 1)
    scatter(cB, 1)
    pltpu.async_copy(pk_hbm.at[pl.ds(pkbase + NFULL * PKT, 2 * KT)], it_, q1)
    drain_rows(cC, 2)
    scatter(cC, 2)
    pltpu.make_async_copy(pk_hbm.at[pl.ds(pkbase + NFULL * PKT, 2 * KT)],
                          it_, q1).wait()
    pltpu.async_copy(feats_hbm.at[it_.at[pl.ds(0, KT)]], rt, s1)
    drain_rows(cD, 0)
    scatter(cD, 0)
    pltpu.make_async_copy(feats_hbm.at[it_.at[pl.ds(0, KT)]], rt, s1).wait()
    pltpu.sync_copy(rt, acc.at[it_.at[pl.ds(KT, KT)]], add=True)

    plsc.subcore_barrier()
    pltpu.sync_copy(acc.at[pl.ds(s * NPT, NPT)], out_hbm.at[c, pl.ds(s * NPT, NPT)])


def _bn(h, g, b, eps=1e-5):
    mu = jnp.mean(h, axis=0, keepdims=True)
    var = jnp.mean((h - mu) * (h - mu), axis=0, keepdims=True)
    return g * (h - mu) * lax.rsqrt(var + eps) + b


def _dot(a, b):
    return jnp.dot(a, b, preferred_element_type=jnp.float32)


def _tc_first_body(x, pa, atom_w, atom_b, b0w, b0b, g0, be0, feats_out):
    a = pa[0, :N, :DE] + pa[1, :N, :DE]
    deg = pa[0, :N, DE:DE + 1] + pa[1, :N, DE:DE + 1]
    h = _dot(a, b0w[...]) + deg * b0b[...] + _dot(x[...], atom_w[...]) + atom_b[...]
    feats_out[...] = _bn(jnp.maximum(h, 0.0), g0[...], be0[...])


_tc_first = pl.pallas_call(
    _tc_first_body,
    out_shape=jax.ShapeDtypeStruct((N, D), jnp.float32),
)


def _tc_layer_body(p, pa, feats, bw, bb, h1w, h1b, h2w, h2b,
                   g1, be1, g2, be2, out):
    a = pa[0, :N, :DE] + pa[1, :N, :DE]
    deg = pa[0, :N, DE:DE + 1] + pa[1, :N, DE:DE + 1]
    h2 = _dot(a, bw[...]) + deg * bb[...]
    h1 = p[0, :N] + p[1, :N]
    t = _dot(h1, h1w[...]) + h1b[...] + h2
    h = _bn(jnp.maximum(t, 0.0), g1[...], be1[...])
    u = _dot(h, h2w[...]) + h2b[...] + feats[...]
    out[...] = _bn(jnp.maximum(u, 0.0), g2[...], be2[...])


_tc_layer = pl.pallas_call(
    _tc_layer_body,
    out_shape=jax.ShapeDtypeStruct((N, D), jnp.float32),
)


def kernel(x, edge_index, edge_attr, params):
    srcs = edge_index[0].astype(jnp.int32)
    dsts = edge_index[1].astype(jnp.int32)
    zeros_nd = jnp.zeros((NP, D), jnp.float32)

    ea_ext = jnp.concatenate(
        [edge_attr, jnp.ones((E, 1), jnp.float32),
         jnp.zeros((E, D - DE - 1), jnp.float32)], axis=1)

    s2 = srcs.reshape(NW, EPT)
    d2 = dsts.reshape(NW, EPT)
    sf = s2[:, :NFULL * KB].reshape(NW, NFULL, KB)
    df = d2[:, :NFULL * KB].reshape(NW, NFULL, KB)
    pf = jnp.stack([sf, df], axis=2).reshape(NW, NFULL * PKT)
    pt = jnp.concatenate([s2[:, NFULL * KB:], d2[:, NFULL * KB:]], axis=1)
    pk = jnp.concatenate([pf, pt], axis=1).reshape(-1)

    pa = _sc_edge_scatter(ea_ext, pk, zeros_nd)

    feats = _tc_first(
        x, pa, params["atom_W"], params["atom_b"],
        params["bond0_W"], params["bond0_b"], params["bn0_g"], params["bn0_b"],
    )

    for lp in params["layers"]:
        p = _sc_gather_scatter(feats, pk, zeros_nd)
        feats = _tc_layer(
            p, pa, feats, lp["bond_W"], lp["bond_b"],
            lp["h1_W"], lp["h1_b"], lp["h2_W"], lp["h2_b"],
            lp["bn1_g"], lp["bn1_b"], lp["bn2_g"], lp["bn2_b"],
        )
    return feats


# speedup vs baseline: 1.0970x; 1.0970x over previous
import functools

import jax
import jax.numpy as jnp
from jax import lax
from jax.experimental import pallas as pl
from jax.experimental.pallas import tpu as pltpu
from jax.experimental.pallas import tpu_sc as plsc

N = 10000
E = 640000
D = 128
DE = 16

NC = 2
NS = 16
NW = NC * NS
EPT = E // NW
KB = 128
NFULL = EPT // KB
KT = EPT - NFULL * KB
NG = NFULL // 2
NG3 = (NFULL - 3) // 3
NP = 10112
NPT = NP // NS
PKT = 2 * KB
TILE_PK = NFULL * PKT + 2 * KT

_mesh = plsc.VectorSubcoreMesh(
    core_axis_name="c", subcore_axis_name="s", num_cores=NC, num_subcores=NS
)


@functools.partial(
    pl.kernel,
    out_type=jax.ShapeDtypeStruct((NC, NP, D), jnp.float32),
    mesh=_mesh,
    scratch_types=[
        pltpu.VMEM((KB,), jnp.int32),
        pltpu.VMEM((KB,), jnp.int32),
        pltpu.VMEM((KT,), jnp.int32),
        pltpu.VMEM((KB, D), jnp.float32),
        pltpu.VMEM((KB, D), jnp.float32),
        pltpu.VMEM((KT, D), jnp.float32),
        pltpu.VMEM_SHARED((NP, D), jnp.float32),
        pltpu.SemaphoreType.DMA,
        pltpu.SemaphoreType.DMA,
        pltpu.SemaphoreType.DMA,
        pltpu.SemaphoreType.DMA,
    ],
)
def _sc_edge_scatter(ea_hbm, pk_hbm, zeros_hbm, out_a, i0, i1, it_,
                     v0, v1, vt, acc, s0, s1, q0, q1):
    c = lax.axis_index("c")
    s = lax.axis_index("s")
    wid = c * NS + s
    pltpu.sync_copy(zeros_hbm.at[pl.ds(s * NPT, NPT)], acc.at[pl.ds(s * NPT, NPT)])
    plsc.subcore_barrier()
    base = wid * EPT
    pkbase = wid * TILE_PK
    ibufs = (i0, i1)
    isems = (q0, q1)
    vbufs = (v0, v1)
    vsems = (s0, s1)

    def fire(ch, b):
        pltpu.async_copy(pk_hbm.at[pl.ds(pkbase + ch * PKT + KB, KB)],
                         ibufs[b], isems[b])
        pltpu.async_copy(ea_hbm.at[pl.ds(base + ch * KB, KB)],
                         vbufs[b], vsems[b])

    def drain(ch, b):
        pltpu.make_async_copy(pk_hbm.at[pl.ds(pkbase + ch * PKT + KB, KB)],
                              ibufs[b], isems[b]).wait()
        pltpu.make_async_copy(ea_hbm.at[pl.ds(base + ch * KB, KB)],
                              vbufs[b], vsems[b]).wait()

    def scatter(ch, b):
        pltpu.sync_copy(vbufs[b], acc.at[ibufs[b]], add=True)

    fire(0, 0)

    def group(gi, carry):
        for b in range(2):
            ch = gi * 2 + b
            fire(ch + 1, 1 - b)
            drain(ch, b)
            scatter(ch, b)
        return carry

    lax.fori_loop(0, NG - 1, group, 0)
    ch0, ch1 = NFULL - 2, NFULL - 1
    fire(ch1, 1)
    drain(ch0, 0)
    scatter(ch0, 0)
    pltpu.async_copy(pk_hbm.at[pl.ds(pkbase + NFULL * PKT + KT, KT)], it_, q0)
    pltpu.async_copy(ea_hbm.at[pl.ds(base + NFULL * KB, KT)], vt, s0)
    drain(ch1, 1)
    scatter(ch1, 1)
    pltpu.make_async_copy(pk_hbm.at[pl.ds(pkbase + NFULL * PKT + KT, KT)],
                          it_, q0).wait()
    pltpu.make_async_copy(ea_hbm.at[pl.ds(base + NFULL * KB, KT)], vt, s0).wait()
    pltpu.sync_copy(vt, acc.at[it_], add=True)

    plsc.subcore_barrier()
    pltpu.sync_copy(acc.at[pl.ds(s * NPT, NPT)], out_a.at[c, pl.ds(s * NPT, NPT)])


@functools.partial(
    pl.kernel,
    out_type=jax.ShapeDtypeStruct((NC, NP, D), jnp.float32),
    mesh=_mesh,
    scratch_types=[
        pltpu.VMEM((PKT,), jnp.int32),
        pltpu.VMEM((PKT,), jnp.int32),
        pltpu.VMEM((PKT,), jnp.int32),
        pltpu.VMEM((KB, D), jnp.float32),
        pltpu.VMEM((KB, D), jnp.float32),
        pltpu.VMEM((KB, D), jnp.float32),
        pltpu.VMEM_SHARED((NP, D), jnp.float32),
        pltpu.SemaphoreType.DMA,
        pltpu.SemaphoreType.DMA,
        pltpu.SemaphoreType.DMA,
        pltpu.SemaphoreType.DMA,
        pltpu.SemaphoreType.DMA,
        pltpu.SemaphoreType.DMA,
    ],
)
def _sc_gather_scatter(feats_hbm, pk_hbm, zeros_hbm, out_hbm,
                       i0, i1, i2, r0, r1, r2, acc,
                       s0, s1, s2, q0, q1, q2):
    c = lax.axis_index("c")
    s = lax.axis_index("s")
    wid = c * NS + s
    pltpu.sync_copy(zeros_hbm.at[pl.ds(s * NPT, NPT)], acc.at[pl.ds(s * NPT, NPT)])
    plsc.subcore_barrier()
    pkbase = wid * TILE_PK
    ibufs = (i0, i1, i2)
    isems = (q0, q1, q2)
    rbufs = (r0, r1, r2)
    rsems = (s0, s1, s2)
    it_ = i0.at[pl.ds(0, 2 * KT)]
    rt = r0.at[pl.ds(0, KT)]

    def fire_idx(ch, b):
        pltpu.async_copy(pk_hbm.at[pl.ds(pkbase + ch * PKT, PKT)],
                         ibufs[b], isems[b])

    def drain_idx(ch, b):
        pltpu.make_async_copy(pk_hbm.at[pl.ds(pkbase + ch * PKT, PKT)],
                              ibufs[b], isems[b]).wait()

    def gather(ch, b):
        pltpu.async_copy(feats_hbm.at[ibufs[b].at[pl.ds(0, KB)]],
                         rbufs[b], rsems[b])

    def drain_rows(ch, b):
        pltpu.make_async_copy(feats_hbm.at[ibufs[b].at[pl.ds(0, KB)]],
                              rbufs[b], rsems[b]).wait()

    def scatter(ch, b):
        pltpu.sync_copy(rbufs[b], acc.at[ibufs[b].at[pl.ds(KB, KB)]], add=True)

    pltpu.async_copy(pk_hbm.at[pl.ds(pkbase, PKT)], i0, q0).wait()
    gather(0, 0)
    fire_idx(1, 1)
    drain_idx(1, 1)
    gather(1, 1)
    fire_idx(2, 2)

    def group(gi, carry):
        for b in range(3):
            t = gi * 3 + b
            drain_idx(t + 2, (b + 2) % 3)
            gather(t + 2, (b + 2) % 3)
            drain_rows(t, b)
            scatter(t, b)
            fire_idx(t + 3, b)
        return carry

    lax.fori_loop(0, NG3, group, 0)
    cA, cB, cC = NFULL - 3, NFULL - 2, NFULL - 1
    drain_idx(cC, 2)
    gather(cC, 2)
    drain_rows(cA, 0)
    scatter(cA, 0)
    pltpu.async_copy(pk_hbm.at[pl.ds(pkbase + NFULL * PKT, 2 * KT)], it_, q0)
    drain_rows(cB, 1)
    scatter(cB, 1)
    pltpu.make_async_copy(pk_hbm.at[pl.ds(pkbase + NFULL * PKT, 2 * KT)],
                          it_, q0).wait()
    pltpu.async_copy(feats_hbm.at[it_.at[pl.ds(0, KT)]], rt, s0)
    drain_rows(cC, 2)
    scatter(cC, 2)
    pltpu.make_async_copy(feats_hbm.at[it_.at[pl.ds(0, KT)]], rt, s0).wait()
    pltpu.sync_copy(rt, acc.at[it_.at[pl.ds(KT, KT)]], add=True)

    plsc.subcore_barrier()
    pltpu.sync_copy(acc.at[pl.ds(s * NPT, NPT)], out_hbm.at[c, pl.ds(s * NPT, NPT)])


def _bn(h, g, b, eps=1e-5):
    mu = jnp.mean(h, axis=0, keepdims=True)
    var = jnp.mean((h - mu) * (h - mu), axis=0, keepdims=True)
    return g * (h - mu) * lax.rsqrt(var + eps) + b


def _dot(a, b):
    return jnp.dot(a, b, preferred_element_type=jnp.float32)


def _tc_first_body(x, pa, atom_w, atom_b, b0w, b0b, g0, be0, feats_out):
    a = pa[0, :N, :DE] + pa[1, :N, :DE]
    deg = pa[0, :N, DE:DE + 1] + pa[1, :N, DE:DE + 1]
    h = _dot(a, b0w[...]) + deg * b0b[...] + _dot(x[...], atom_w[...]) + atom_b[...]
    feats_out[...] = _bn(jnp.maximum(h, 0.0), g0[...], be0[...])


_tc_first = pl.pallas_call(
    _tc_first_body,
    out_shape=jax.ShapeDtypeStruct((N, D), jnp.float32),
)


def _tc_layer_body(p, pa, feats, bw, bb, h1w, h1b, h2w, h2b,
                   g1, be1, g2, be2, out):
    a = pa[0, :N, :DE] + pa[1, :N, :DE]
    deg = pa[0, :N, DE:DE + 1] + pa[1, :N, DE:DE + 1]
    h2 = _dot(a, bw[...]) + deg * bb[...]
    h1 = p[0, :N] + p[1, :N]
    t = _dot(h1, h1w[...]) + h1b[...] + h2
    h = _bn(jnp.maximum(t, 0.0), g1[...], be1[...])
    u = _dot(h, h2w[...]) + h2b[...] + feats[...]
    out[...] = _bn(jnp.maximum(u, 0.0), g2[...], be2[...])


_tc_layer = pl.pallas_call(
    _tc_layer_body,
    out_shape=jax.ShapeDtypeStruct((N, D), jnp.float32),
)


def kernel(x, edge_index, edge_attr, params):
    srcs = edge_index[0].astype(jnp.int32)
    dsts = edge_index[1].astype(jnp.int32)
    zeros_nd = jnp.zeros((NP, D), jnp.float32)

    ea_ext = jnp.concatenate(
        [edge_attr, jnp.ones((E, 1), jnp.float32),
         jnp.zeros((E, D - DE - 1), jnp.float32)], axis=1)

    s2 = srcs.reshape(NW, EPT)
    d2 = dsts.reshape(NW, EPT)
    sf = s2[:, :NFULL * KB].reshape(NW, NFULL, KB)
    df = d2[:, :NFULL * KB].reshape(NW, NFULL, KB)
    pf = jnp.stack([sf, df], axis=2).reshape(NW, NFULL * PKT)
    pt = jnp.concatenate([s2[:, NFULL * KB:], d2[:, NFULL * KB:]], axis=1)
    pk = jnp.concatenate([pf, pt], axis=1).reshape(-1)

    pa = _sc_edge_scatter(ea_ext, pk, zeros_nd)

    feats = _tc_first(
        x, pa, params["atom_W"], params["atom_b"],
        params["bond0_W"], params["bond0_b"], params["bn0_g"], params["bn0_b"],
    )

    for lp in params["layers"]:
        p = _sc_gather_scatter(feats, pk, zeros_nd)
        feats = _tc_layer(
            p, pa, feats, lp["bond_W"], lp["bond_b"],
            lp["h1_W"], lp["h1_b"], lp["h2_W"], lp["h2_b"],
            lp["bn1_g"], lp["bn1_b"], lp["bn2_g"], lp["bn2_b"],
        )
    return feats


# speedup vs baseline: 1.0974x; 1.0004x over previous
import functools

import jax
import jax.numpy as jnp
from jax import lax
from jax.experimental import pallas as pl
from jax.experimental.pallas import tpu as pltpu
from jax.experimental.pallas import tpu_sc as plsc

N = 10000
E = 640000
D = 128
DE = 16

NC = 2
NS = 16
NW = NC * NS
EPT = E // NW
KB = 128
NFULL = EPT // KB
KT = EPT - NFULL * KB
NG = NFULL // 2
NG3 = (NFULL - 3) // 3
NP = 10112
NPT = NP // NS
PKT = 2 * KB
TILE_PK = NFULL * PKT + 2 * KT

_mesh = plsc.VectorSubcoreMesh(
    core_axis_name="c", subcore_axis_name="s", num_cores=NC, num_subcores=NS
)


@functools.partial(
    pl.kernel,
    out_type=jax.ShapeDtypeStruct((NC, NP, D), jnp.float32),
    mesh=_mesh,
    scratch_types=[
        pltpu.VMEM((KB,), jnp.int32),
        pltpu.VMEM((KB,), jnp.int32),
        pltpu.VMEM((KT,), jnp.int32),
        pltpu.VMEM((KB, D), jnp.float32),
        pltpu.VMEM((KB, D), jnp.float32),
        pltpu.VMEM((KT, D), jnp.float32),
        pltpu.VMEM_SHARED((NP, D), jnp.float32),
        pltpu.SemaphoreType.DMA,
        pltpu.SemaphoreType.DMA,
        pltpu.SemaphoreType.DMA,
        pltpu.SemaphoreType.DMA,
    ],
)
def _sc_edge_scatter(ea_hbm, pk_hbm, zeros_hbm, out_a, i0, i1, it_,
                     v0, v1, vt, acc, s0, s1, q0, q1):
    c = lax.axis_index("c")
    s = lax.axis_index("s")
    wid = c * NS + s
    pltpu.sync_copy(zeros_hbm.at[pl.ds(s * NPT, NPT)], acc.at[pl.ds(s * NPT, NPT)])
    plsc.subcore_barrier()
    base = wid * EPT
    pkbase = wid * TILE_PK
    ibufs = (i0, i1)
    isems = (q0, q1)
    vbufs = (v0, v1)
    vsems = (s0, s1)

    def fire(ch, b):
        pltpu.async_copy(pk_hbm.at[pl.ds(pkbase + ch * PKT + KB, KB)],
                         ibufs[b], isems[b])
        pltpu.async_copy(ea_hbm.at[pl.ds(base + ch * KB, KB)],
                         vbufs[b], vsems[b])

    def drain(ch, b):
        pltpu.make_async_copy(pk_hbm.at[pl.ds(pkbase + ch * PKT + KB, KB)],
                              ibufs[b], isems[b]).wait()
        pltpu.make_async_copy(ea_hbm.at[pl.ds(base + ch * KB, KB)],
                              vbufs[b], vsems[b]).wait()

    def scatter(ch, b):
        pltpu.sync_copy(vbufs[b], acc.at[ibufs[b]], add=True)

    fire(0, 0)

    def group(gi, carry):
        for b in range(2):
            ch = gi * 2 + b
            fire(ch + 1, 1 - b)
            drain(ch, b)
            scatter(ch, b)
        return carry

    lax.fori_loop(0, NG - 1, group, 0)
    ch0, ch1 = NFULL - 2, NFULL - 1
    fire(ch1, 1)
    drain(ch0, 0)
    scatter(ch0, 0)
    pltpu.async_copy(pk_hbm.at[pl.ds(pkbase + NFULL * PKT + KT, KT)], it_, q0)
    pltpu.async_copy(ea_hbm.at[pl.ds(base + NFULL * KB, KT)], vt, s0)
    drain(ch1, 1)
    scatter(ch1, 1)
    pltpu.make_async_copy(pk_hbm.at[pl.ds(pkbase + NFULL * PKT + KT, KT)],
                          it_, q0).wait()
    pltpu.make_async_copy(ea_hbm.at[pl.ds(base + NFULL * KB, KT)], vt, s0).wait()
    pltpu.sync_copy(vt, acc.at[it_], add=True)

    plsc.subcore_barrier()
    pltpu.sync_copy(acc.at[pl.ds(s * NPT, NPT)], out_a.at[c, pl.ds(s * NPT, NPT)])


@functools.partial(
    pl.kernel,
    out_type=jax.ShapeDtypeStruct((NC, NP, D), jnp.float32),
    mesh=_mesh,
    scratch_types=[
        pltpu.VMEM((PKT,), jnp.int32),
        pltpu.VMEM((PKT,), jnp.int32),
        pltpu.VMEM((PKT,), jnp.int32),
        pltpu.VMEM((KB, D), jnp.float32),
        pltpu.VMEM((KB, D), jnp.float32),
        pltpu.VMEM((KB, D), jnp.float32),
        pltpu.VMEM_SHARED((NP, D), jnp.float32),
        pltpu.SemaphoreType.DMA,
        pltpu.SemaphoreType.DMA,
        pltpu.SemaphoreType.DMA,
        pltpu.SemaphoreType.DMA,
        pltpu.SemaphoreType.DMA,
        pltpu.SemaphoreType.DMA,
    ],
)
def _sc_gather_scatter(feats_hbm, pk_hbm, zeros_hbm, out_hbm,
                       i0, i1, i2, r0, r1, r2, acc,
                       s0, s1, s2, q0, q1, q2):
    c = lax.axis_index("c")
    s = lax.axis_index("s")
    wid = c * NS + s
    pltpu.sync_copy(zeros_hbm.at[pl.ds(s * NPT, NPT)], acc.at[pl.ds(s * NPT, NPT)])
    plsc.subcore_barrier()
    pkbase = wid * TILE_PK
    ibufs = (i0, i1, i2)
    isems = (q0, q1, q2)
    rbufs = (r0, r1, r2)
    rsems = (s0, s1, s2)
    it_ = i0.at[pl.ds(0, 2 * KT)]
    rt = r0.at[pl.ds(0, KT)]

    def fire_idx(ch, b):
        pltpu.async_copy(pk_hbm.at[pl.ds(pkbase + ch * PKT, PKT)],
                         ibufs[b], isems[b])

    def drain_idx(ch, b):
        pltpu.make_async_copy(pk_hbm.at[pl.ds(pkbase + ch * PKT, PKT)],
                              ibufs[b], isems[b]).wait()

    def gather(ch, b):
        pltpu.async_copy(feats_hbm.at[ibufs[b].at[pl.ds(0, KB)]],
                         rbufs[b], rsems[b])

    def drain_rows(ch, b):
        pltpu.make_async_copy(feats_hbm.at[ibufs[b].at[pl.ds(0, KB)]],
                              rbufs[b], rsems[b]).wait()

    def scatter(ch, b):
        pltpu.sync_copy(rbufs[b], acc.at[ibufs[b].at[pl.ds(KB, KB)]], add=True)

    pltpu.async_copy(pk_hbm.at[pl.ds(pkbase, PKT)], i0, q0).wait()
    gather(0, 0)
    fire_idx(1, 1)
    drain_idx(1, 1)
    gather(1, 1)
    fire_idx(2, 2)

    def group(gi, carry):
        for b in range(3):
            t = gi * 3 + b
            drain_idx(t + 2, (b + 2) % 3)
            gather(t + 2, (b + 2) % 3)
            drain_rows(t, b)
            scatter(t, b)
            fire_idx(t + 3, b)
        return carry

    lax.fori_loop(0, NG3, group, 0)
    cA, cB, cC = NFULL - 3, NFULL - 2, NFULL - 1
    drain_idx(cC, 2)
    gather(cC, 2)
    drain_rows(cA, 0)
    scatter(cA, 0)
    pltpu.async_copy(pk_hbm.at[pl.ds(pkbase + NFULL * PKT, 2 * KT)], it_, q0)
    drain_rows(cB, 1)
    scatter(cB, 1)
    pltpu.make_async_copy(pk_hbm.at[pl.ds(pkbase + NFULL * PKT, 2 * KT)],
                          it_, q0).wait()
    pltpu.async_copy(feats_hbm.at[it_.at[pl.ds(0, KT)]], rt, s0)
    drain_rows(cC, 2)
    scatter(cC, 2)
    pltpu.make_async_copy(feats_hbm.at[it_.at[pl.ds(0, KT)]], rt, s0).wait()
    pltpu.sync_copy(rt, acc.at[it_.at[pl.ds(KT, KT)]], add=True)

    plsc.subcore_barrier()
    pltpu.sync_copy(acc.at[pl.ds(s * NPT, NPT)], out_hbm.at[c, pl.ds(s * NPT, NPT)])


def _bn(h, g, b, eps=1e-5):
    mu = jnp.mean(h, axis=0, keepdims=True)
    var = jnp.mean((h - mu) * (h - mu), axis=0, keepdims=True)
    return g * (h - mu) * lax.rsqrt(var + eps) + b


def _dot(a, b):
    return jnp.dot(a, b, preferred_element_type=jnp.float32)


def _tc_first_body(x, pa, atom_w, atom_b, b0w, b0b, g0, be0, feats_out):
    a = pa[0, :N, :DE] + pa[1, :N, :DE]
    deg = pa[0, :N, DE:DE + 1] + pa[1, :N, DE:DE + 1]
    h = _dot(a, b0w[...]) + deg * b0b[...] + _dot(x[...], atom_w[...]) + atom_b[...]
    feats_out[...] = _bn(jnp.maximum(h, 0.0), g0[...], be0[...])


_tc_first = pl.pallas_call(
    _tc_first_body,
    out_shape=jax.ShapeDtypeStruct((N, D), jnp.float32),
)


def _tc_bond_body(pa, bw, bb, out):
    a = pa[0, :N, :DE] + pa[1, :N, :DE]
    deg = pa[0, :N, DE:DE + 1] + pa[1, :N, DE:DE + 1]
    out[...] = _dot(a, bw[...]) + deg * bb[...]


_tc_bond = pl.pallas_call(
    _tc_bond_body,
    out_shape=jax.ShapeDtypeStruct((N, D), jnp.float32),
)


def _tc_layer_body(p, h2ref, feats, h1w, h1b, h2w, h2b,
                   g1, be1, g2, be2, out):
    h1 = p[0, :N] + p[1, :N]
    t = _dot(h1, h1w[...]) + h1b[...] + h2ref[...]
    h = _bn(jnp.maximum(t, 0.0), g1[...], be1[...])
    u = _dot(h, h2w[...]) + h2b[...] + feats[...]
    out[...] = _bn(jnp.maximum(u, 0.0), g2[...], be2[...])


_tc_layer = pl.pallas_call(
    _tc_layer_body,
    out_shape=jax.ShapeDtypeStruct((N, D), jnp.float32),
)


def kernel(x, edge_index, edge_attr, params):
    srcs = edge_index[0].astype(jnp.int32)
    dsts = edge_index[1].astype(jnp.int32)
    zeros_nd = jnp.zeros((NP, D), jnp.float32)

    ea_ext = jnp.concatenate(
        [edge_attr, jnp.ones((E, 1), jnp.float32),
         jnp.zeros((E, D - DE - 1), jnp.float32)], axis=1)

    s2 = srcs.reshape(NW, EPT)
    d2 = dsts.reshape(NW, EPT)
    sf = s2[:, :NFULL * KB].reshape(NW, NFULL, KB)
    df = d2[:, :NFULL * KB].reshape(NW, NFULL, KB)
    pf = jnp.stack([sf, df], axis=2).reshape(NW, NFULL * PKT)
    pt = jnp.concatenate([s2[:, NFULL * KB:], d2[:, NFULL * KB:]], axis=1)
    pk = jnp.concatenate([pf, pt], axis=1).reshape(-1)

    pa = _sc_edge_scatter(ea_ext, pk, zeros_nd)

    feats = _tc_first(
        x, pa, params["atom_W"], params["atom_b"],
        params["bond0_W"], params["bond0_b"], params["bn0_g"], params["bn0_b"],
    )

    h2s = [_tc_bond(pa, lp["bond_W"], lp["bond_b"]) for lp in params["layers"]]

    for lp, h2 in zip(params["layers"], h2s):
        p = _sc_gather_scatter(feats, pk, zeros_nd)
        feats = _tc_layer(
            p, h2, feats,
            lp["h1_W"], lp["h1_b"], lp["h2_W"], lp["h2_b"],
            lp["bn1_g"], lp["bn1_b"], lp["bn2_g"], lp["bn2_b"],
        )
    return feats


# speedup vs baseline: 1.0993x; 1.0017x over previous
import functools

import jax
import jax.numpy as jnp
from jax import lax
from jax.experimental import pallas as pl
from jax.experimental.pallas import tpu as pltpu
from jax.experimental.pallas import tpu_sc as plsc

N = 10000
E = 640000
D = 128
DE = 16

NC = 2
NS = 16
NW = NC * NS
EPT = E // NW
KB = 128
NFULL = EPT // KB
KT = EPT - NFULL * KB
NG = NFULL // 2
NG3 = (NFULL - 3) // 3
NP = 10112
NPT = NP // NS
PKT = 2 * KB
TILE_PK = NFULL * PKT + 2 * KT

_mesh = plsc.VectorSubcoreMesh(
    core_axis_name="c", subcore_axis_name="s", num_cores=NC, num_subcores=NS
)


@functools.partial(
    pl.kernel,
    out_type=jax.ShapeDtypeStruct((NC, NP, D), jnp.float32),
    mesh=_mesh,
    scratch_types=[
        pltpu.VMEM((KB,), jnp.int32),
        pltpu.VMEM((KB,), jnp.int32),
        pltpu.VMEM((KT,), jnp.int32),
        pltpu.VMEM((KB, D), jnp.float32),
        pltpu.VMEM((KB, D), jnp.float32),
        pltpu.VMEM((KT, D), jnp.float32),
        pltpu.VMEM_SHARED((NP, D), jnp.float32),
        pltpu.SemaphoreType.DMA,
        pltpu.SemaphoreType.DMA,
        pltpu.SemaphoreType.DMA,
        pltpu.SemaphoreType.DMA,
    ],
)
def _sc_edge_scatter(ea_hbm, pk_hbm, zeros_hbm, out_a, i0, i1, it_,
                     v0, v1, vt, acc, s0, s1, q0, q1):
    c = lax.axis_index("c")
    s = lax.axis_index("s")
    wid = c * NS + s
    pltpu.sync_copy(zeros_hbm.at[pl.ds(s * NPT, NPT)], acc.at[pl.ds(s * NPT, NPT)])
    plsc.subcore_barrier()
    base = wid * EPT
    pkbase = wid * TILE_PK
    ibufs = (i0, i1)
    isems = (q0, q1)
    vbufs = (v0, v1)
    vsems = (s0, s1)

    def fire(ch, b):
        pltpu.async_copy(pk_hbm.at[pl.ds(pkbase + ch * PKT + KB, KB)],
                         ibufs[b], isems[b])
        pltpu.async_copy(ea_hbm.at[pl.ds(base + ch * KB, KB)],
                         vbufs[b], vsems[b])

    def drain(ch, b):
        pltpu.make_async_copy(pk_hbm.at[pl.ds(pkbase + ch * PKT + KB, KB)],
                              ibufs[b], isems[b]).wait()
        pltpu.make_async_copy(ea_hbm.at[pl.ds(base + ch * KB, KB)],
                              vbufs[b], vsems[b]).wait()

    def scatter(ch, b):
        pltpu.sync_copy(vbufs[b], acc.at[ibufs[b]], add=True)

    fire(0, 0)

    def group(gi, carry):
        for b in range(2):
            ch = gi * 2 + b
            fire(ch + 1, 1 - b)
            drain(ch, b)
            scatter(ch, b)
        return carry

    lax.fori_loop(0, NG - 1, group, 0)
    ch0, ch1 = NFULL - 2, NFULL - 1
    fire(ch1, 1)
    drain(ch0, 0)
    scatter(ch0, 0)
    pltpu.async_copy(pk_hbm.at[pl.ds(pkbase + NFULL * PKT + KT, KT)], it_, q0)
    pltpu.async_copy(ea_hbm.at[pl.ds(base + NFULL * KB, KT)], vt, s0)
    drain(ch1, 1)
    scatter(ch1, 1)
    pltpu.make_async_copy(pk_hbm.at[pl.ds(pkbase + NFULL * PKT + KT, KT)],
                          it_, q0).wait()
    pltpu.make_async_copy(ea_hbm.at[pl.ds(base + NFULL * KB, KT)], vt, s0).wait()
    pltpu.sync_copy(vt, acc.at[it_], add=True)

    plsc.subcore_barrier()
    pltpu.sync_copy(acc.at[pl.ds(s * NPT, NPT)], out_a.at[c, pl.ds(s * NPT, NPT)])


@functools.partial(
    pl.kernel,
    out_type=jax.ShapeDtypeStruct((NC, NP, D), jnp.float32),
    mesh=_mesh,
    scratch_types=[
        pltpu.VMEM((PKT,), jnp.int32),
        pltpu.VMEM((PKT,), jnp.int32),
        pltpu.VMEM((PKT,), jnp.int32),
        pltpu.VMEM((KB, D), jnp.float32),
        pltpu.VMEM((KB, D), jnp.float32),
        pltpu.VMEM((KB, D), jnp.float32),
        pltpu.VMEM_SHARED((NP, D), jnp.float32),
        pltpu.SemaphoreType.DMA,
        pltpu.SemaphoreType.DMA,
        pltpu.SemaphoreType.DMA,
        pltpu.SemaphoreType.DMA,
        pltpu.SemaphoreType.DMA,
        pltpu.SemaphoreType.DMA,
    ],
)
def _sc_gather_scatter(feats_hbm, pk_hbm, zeros_hbm, out_hbm,
                       i0, i1, i2, r0, r1, r2, acc,
                       s0, s1, s2, q0, q1, q2):
    c = lax.axis_index("c")
    s = lax.axis_index("s")
    wid = c * NS + s
    pltpu.sync_copy(zeros_hbm.at[pl.ds(s * NPT, NPT)], acc.at[pl.ds(s * NPT, NPT)])
    plsc.subcore_barrier()
    pkbase = wid * TILE_PK
    ibufs = (i0, i1, i2)
    isems = (q0, q1, q2)
    rbufs = (r0, r1, r2)
    rsems = (s0, s1, s2)
    it_ = i0.at[pl.ds(0, 2 * KT)]
    rt = r0.at[pl.ds(0, KT)]

    def fire_idx(ch, b):
        pltpu.async_copy(pk_hbm.at[pl.ds(pkbase + ch * PKT, PKT)],
                         ibufs[b], isems[b])

    def drain_idx(ch, b):
        pltpu.make_async_copy(pk_hbm.at[pl.ds(pkbase + ch * PKT, PKT)],
                              ibufs[b], isems[b]).wait()

    def gather(ch, b):
        pltpu.async_copy(feats_hbm.at[ibufs[b].at[pl.ds(0, KB)]],
                         rbufs[b], rsems[b])

    def drain_rows(ch, b):
        pltpu.make_async_copy(feats_hbm.at[ibufs[b].at[pl.ds(0, KB)]],
                              rbufs[b], rsems[b]).wait()

    def scatter(ch, b):
        pltpu.sync_copy(rbufs[b], acc.at[ibufs[b].at[pl.ds(KB, KB)]], add=True)

    pltpu.async_copy(pk_hbm.at[pl.ds(pkbase, PKT)], i0, q0).wait()
    gather(0, 0)
    fire_idx(1, 1)
    drain_idx(1, 1)
    gather(1, 1)
    fire_idx(2, 2)

    def group(gi, carry):
        for b in range(3):
            t = gi * 3 + b
            drain_idx(t + 2, (b + 2) % 3)
            gather(t + 2, (b + 2) % 3)
            drain_rows(t, b)
            scatter(t, b)
            fire_idx(t + 3, b)
        return carry

    lax.fori_loop(0, NG3, group, 0)
    cA, cB, cC = NFULL - 3, NFULL - 2, NFULL - 1
    drain_idx(cC, 2)
    gather(cC, 2)
    drain_rows(cA, 0)
    scatter(cA, 0)
    pltpu.async_copy(pk_hbm.at[pl.ds(pkbase + NFULL * PKT, 2 * KT)], it_, q0)
    drain_rows(cB, 1)
    scatter(cB, 1)
    pltpu.make_async_copy(pk_hbm.at[pl.ds(pkbase + NFULL * PKT, 2 * KT)],
                          it_, q0).wait()
    pltpu.async_copy(feats_hbm.at[it_.at[pl.ds(0, KT)]], rt, s0)
    drain_rows(cC, 2)
    scatter(cC, 2)
    pltpu.make_async_copy(feats_hbm.at[it_.at[pl.ds(0, KT)]], rt, s0).wait()
    pltpu.sync_copy(rt, acc.at[it_.at[pl.ds(KT, KT)]], add=True)

    plsc.subcore_barrier()
    pltpu.sync_copy(acc.at[pl.ds(s * NPT, NPT)], out_hbm.at[c, pl.ds(s * NPT, NPT)])


def _bn(h, g, b, eps=1e-5):
    mu = jnp.mean(h, axis=0, keepdims=True)
    var = jnp.mean((h - mu) * (h - mu), axis=0, keepdims=True)
    return g * (h - mu) * lax.rsqrt(var + eps) + b


def _dot(a, b):
    return jnp.dot(a, b, preferred_element_type=jnp.float32)


def _tc_atom_body(x, atom_w, atom_b, out):
    out[...] = _dot(x[...], atom_w[...]) + atom_b[...]


_tc_atom = pl.pallas_call(
    _tc_atom_body,
    out_shape=jax.ShapeDtypeStruct((N, D), jnp.float32),
)


def _tc_first_body(xa, pa, b0w, b0b, g0, be0, feats_out):
    a = pa[0, :N, :DE] + pa[1, :N, :DE]
    deg = pa[0, :N, DE:DE + 1] + pa[1, :N, DE:DE + 1]
    h = _dot(a, b0w[...]) + deg * b0b[...] + xa[...]
    feats_out[...] = _bn(jnp.maximum(h, 0.0), g0[...], be0[...])


_tc_first = pl.pallas_call(
    _tc_first_body,
    out_shape=jax.ShapeDtypeStruct((N, D), jnp.float32),
)


def _tc_bond_body(pa, bw, bb, out):
    a = pa[0, :N, :DE] + pa[1, :N, :DE]
    deg = pa[0, :N, DE:DE + 1] + pa[1, :N, DE:DE + 1]
    out[...] = _dot(a, bw[...]) + deg * bb[...]


_tc_bond = pl.pallas_call(
    _tc_bond_body,
    out_shape=jax.ShapeDtypeStruct((N, D), jnp.float32),
)


def _tc_layer_body(p, h2ref, feats, h1w, h1b, h2w, h2b,
                   g1, be1, g2, be2, out):
    h1 = p[0, :N] + p[1, :N]
    t = _dot(h1, h1w[...]) + h1b[...] + h2ref[...]
    h = _bn(jnp.maximum(t, 0.0), g1[...], be1[...])
    u = _dot(h, h2w[...]) + h2b[...] + feats[...]
    out[...] = _bn(jnp.maximum(u, 0.0), g2[...], be2[...])


_tc_layer = pl.pallas_call(
    _tc_layer_body,
    out_shape=jax.ShapeDtypeStruct((N, D), jnp.float32),
)


def kernel(x, edge_index, edge_attr, params):
    srcs = edge_index[0].astype(jnp.int32)
    dsts = edge_index[1].astype(jnp.int32)
    zeros_nd = jnp.zeros((NP, D), jnp.float32)

    ea_ext = jnp.concatenate(
        [edge_attr, jnp.ones((E, 1), jnp.float32),
         jnp.zeros((E, D - DE - 1), jnp.float32)], axis=1)

    s2 = srcs.reshape(NW, EPT)
    d2 = dsts.reshape(NW, EPT)
    sf = s2[:, :NFULL * KB].reshape(NW, NFULL, KB)
    df = d2[:, :NFULL * KB].reshape(NW, NFULL, KB)
    pf = jnp.stack([sf, df], axis=2).reshape(NW, NFULL * PKT)
    pt = jnp.concatenate([s2[:, NFULL * KB:], d2[:, NFULL * KB:]], axis=1)
    pk = jnp.concatenate([pf, pt], axis=1).reshape(-1)

    xa = _tc_atom(x, params["atom_W"], params["atom_b"])
    pa = _sc_edge_scatter(ea_ext, pk, zeros_nd)

    feats = _tc_first(
        xa, pa,
        params["bond0_W"], params["bond0_b"], params["bn0_g"], params["bn0_b"],
    )

    h2s = [_tc_bond(pa, lp["bond_W"], lp["bond_b"]) for lp in params["layers"]]

    for lp, h2 in zip(params["layers"], h2s):
        p = _sc_gather_scatter(feats, pk, zeros_nd)
        feats = _tc_layer(
            p, h2, feats,
            lp["h1_W"], lp["h1_b"], lp["h2_W"], lp["h2_b"],
            lp["bn1_g"], lp["bn1_b"], lp["bn2_g"], lp["bn2_b"],
        )
    return feats
